# baseline (speedup 1.0000x reference)
import jax
import jax.numpy as jnp
from jax import lax
from jax.experimental import pallas as pl
from jax.experimental.pallas import tpu as pltpu

B = 16384
D = 1024
POS_WEIGHT = 2.0
EPS = 1e-8
K = 20
ROWS_PER_BLOCK = 1024
NUM_BLOCKS = B // ROWS_PER_BLOCK
SEL_ROWS = 128
SEL_COLS = B // SEL_ROWS
BIG = 1 << 30
TILES = ROWS_PER_BLOCK // SEL_COLS


def _fused_kernel(a_ref, b_ref, lab_ref, o_ref, sim_acc):
    i = pl.program_id(0)
    a = a_ref[...]
    b = b_ref[...]
    num = jnp.sum(a * b, axis=1, keepdims=True)
    na = jnp.sqrt(jnp.sum(a * a, axis=1, keepdims=True))
    nb = jnp.sqrt(jnp.sum(b * b, axis=1, keepdims=True))
    sim = num / jnp.maximum(na * nb, EPS)
    sim_acc[pl.ds(TILES * i, TILES), :] = sim.reshape(TILES, SEL_COLS)

    @pl.when(i == NUM_BLOCKS - 1)
    def _select():
        simf = sim_acc[...]
        lab = lab_ref[...]
        neg = jnp.where(lab == 0.0, simf, -3.0)
        pos = jnp.where(lab == 0.0, 3.0, simf)
        r = lax.broadcasted_iota(jnp.int32, (SEL_ROWS, SEL_COLS), 0)
        c = lax.broadcasted_iota(jnp.int32, (SEL_ROWS, SEL_COLS), 1)
        flat = r * SEL_COLS + c

        vn, vp = neg, pos
        tot_n = jnp.float32(0.0)
        tot_p = jnp.float32(0.0)
        rem_n = jnp.float32(K)
        rem_p = jnp.float32(K)
        one = jnp.ones((SEL_ROWS, SEL_COLS), jnp.float32)
        zero = jnp.zeros((SEL_ROWS, SEL_COLS), jnp.float32)
        for _ in range(K):
            mn = jnp.max(vn)
            mp = jnp.min(vp)
            eq_n = vn == mn
            eq_p = vp == mp
            vn = jnp.where(eq_n, -3.0, vn)
            vp = jnp.where(eq_p, 3.0, vp)
            cnt_n = jnp.sum(jnp.where(eq_n, one, zero))
            cnt_p = jnp.sum(jnp.where(eq_p, one, zero))
            take_n = jnp.maximum(jnp.minimum(cnt_n, rem_n), 0.0)
            take_p = jnp.maximum(jnp.minimum(cnt_p, rem_p), 0.0)
            rem_n = rem_n - cnt_n
            rem_p = rem_p - cnt_p
            dp = mp - 1.0
            tot_n = tot_n + take_n * mn * mn
            tot_p = tot_p + take_p * dp * dp
        o_ref[...] = jnp.broadcast_to((tot_n + tot_p) * (1.0 / (2 * K)), (1, 1))


def kernel(sample_1, sample_2, labels, original_target):
    lab2d = labels.reshape(SEL_ROWS, SEL_COLS)
    out = pl.pallas_call(
        _fused_kernel,
        grid=(NUM_BLOCKS,),
        in_specs=[
            pl.BlockSpec((ROWS_PER_BLOCK, D), lambda i: (i, 0)),
            pl.BlockSpec((ROWS_PER_BLOCK, D), lambda i: (i, 0)),
            pl.BlockSpec((SEL_ROWS, SEL_COLS), lambda i: (0, 0)),
        ],
        out_specs=pl.BlockSpec((1, 1), lambda i: (0, 0)),
        out_shape=jax.ShapeDtypeStruct((1, 1), jnp.float32),
        scratch_shapes=[pltpu.VMEM((SEL_ROWS, SEL_COLS), jnp.float32)],
    )(sample_1, sample_2, lab2d)

    weight = (POS_WEIGHT - 1.0) * jnp.float32(original_target) + 1.0
    return out[0, 0] * weight


# speedup vs baseline: 1.0004x; 1.0004x over previous
import jax
import jax.numpy as jnp
from jax.experimental import pallas as pl
from jax.experimental.pallas import tpu as pltpu

B = 16384
D = 1024
POS_WEIGHT = 2.0
EPS = 1e-8
K = 20
ROWS_PER_BLOCK = 1024
NUM_BLOCKS = B // ROWS_PER_BLOCK
SEL_ROWS = 128
SEL_COLS = B // SEL_ROWS
TILES = ROWS_PER_BLOCK // SEL_COLS


def _fused_kernel(a_ref, b_ref, lab_ref, o_ref, sim_acc):
    i = pl.program_id(0)
    a = a_ref[...]
    b = b_ref[...]
    num = jnp.sum(a * b, axis=1, keepdims=True)
    na = jnp.sqrt(jnp.sum(a * a, axis=1, keepdims=True))
    nb = jnp.sqrt(jnp.sum(b * b, axis=1, keepdims=True))
    sim = num / jnp.maximum(na * nb, EPS)
    sim_acc[pl.ds(TILES * i, TILES), :] = sim.reshape(TILES, SEL_COLS)

    @pl.when(i == NUM_BLOCKS - 1)
    def _select():
        simf = sim_acc[...]
        lab = lab_ref[...]
        vn = jnp.where(lab == 0.0, simf, -3.0)
        vp = jnp.where(lab == 0.0, 3.0, simf)
        tot_n = jnp.float32(0.0)
        tot_p = jnp.float32(0.0)
        rem_n = jnp.float32(K)
        rem_p = jnp.float32(K)
        one = jnp.ones((SEL_ROWS, SEL_COLS), jnp.float32)
        zero = jnp.zeros((SEL_ROWS, SEL_COLS), jnp.float32)
        for _ in range(K):
            mn = jnp.max(vn)
            mp = jnp.min(vp)
            eq_n = vn == mn
            eq_p = vp == mp
            vn = jnp.where(eq_n, -3.0, vn)
            vp = jnp.where(eq_p, 3.0, vp)
            cnt_n = jnp.sum(jnp.where(eq_n, one, zero))
            cnt_p = jnp.sum(jnp.where(eq_p, one, zero))
            take_n = jnp.maximum(jnp.minimum(cnt_n, rem_n), 0.0)
            take_p = jnp.maximum(jnp.minimum(cnt_p, rem_p), 0.0)
            rem_n = rem_n - cnt_n
            rem_p = rem_p - cnt_p
            dp = mp - 1.0
            tot_n = tot_n + take_n * mn * mn
            tot_p = tot_p + take_p * dp * dp
        o_ref[...] = jnp.broadcast_to((tot_n + tot_p) * (1.0 / (2 * K)), (1, 1))


def kernel(sample_1, sample_2, labels, original_target):
    lab2d = labels.reshape(SEL_ROWS, SEL_COLS)
    out = pl.pallas_call(
        _fused_kernel,
        grid=(NUM_BLOCKS,),
        in_specs=[
            pl.BlockSpec((ROWS_PER_BLOCK, D), lambda i: (i, 0)),
            pl.BlockSpec((ROWS_PER_BLOCK, D), lambda i: (i, 0)),
            pl.BlockSpec((SEL_ROWS, SEL_COLS), lambda i: (0, 0)),
        ],
        out_specs=pl.BlockSpec((1, 1), lambda i: (0, 0)),
        out_shape=jax.ShapeDtypeStruct((1, 1), jnp.float32),
        scratch_shapes=[pltpu.VMEM((SEL_ROWS, SEL_COLS), jnp.float32)],
    )(sample_1, sample_2, lab2d)

    weight = (POS_WEIGHT - 1.0) * jnp.float32(original_target) + 1.0
    return out[0, 0] * weight
